# baseline (speedup 1.0000x reference)
import functools

import jax
import jax.numpy as jnp
from jax import lax
from jax.experimental import pallas as pl
from jax.experimental.pallas import tpu as pltpu
from jax.experimental.pallas import tpu_sc as plsc

_SC_INFO = plsc.get_sparse_core_info()
_NC = _SC_INFO.num_cores
_NS = _SC_INFO.num_subcores
_NW = _NC * _NS


_SEQ, _D = 8192, 2048
_ROWS_PER_W = _SEQ // _NW
_C = 24
_CHUNKS = []
_off = 0
while _off < _ROWS_PER_W:
    _sz = min(_C, _ROWS_PER_W - _off)
    _CHUNKS.append((_off, _sz))
    _off += _sz
_NCH = len(_CHUNKS)
_NBUF = 2


def _sc_copy_body(enc_hbm, out_hbm, *scratch):
    bufs = scratch[:_NBUF]
    gsems = scratch[_NBUF:2 * _NBUF]
    ssems = scratch[2 * _NBUF:3 * _NBUF]
    wid = lax.axis_index("s") * _NC + lax.axis_index("c")
    base = wid * _ROWS_PER_W

    def start_gather(g):
        off, sz = _CHUNKS[g]
        return pltpu.async_copy(
            enc_hbm.at[pl.ds(base + off, sz)],
            bufs[g % _NBUF].at[pl.ds(0, sz)],
            gsems[g % _NBUF],
        )

    def start_scatter(g):
        off, sz = _CHUNKS[g]
        return pltpu.async_copy(
            bufs[g % _NBUF].at[pl.ds(0, sz)],
            out_hbm.at[pl.ds(base + off, sz)],
            ssems[g % _NBUF],
        )

    scat = [None] * _NBUF
    gat = [None] * _NBUF
    for g in range(min(_NBUF, _NCH)):
        gat[g % _NBUF] = start_gather(g)
    for g in range(_NCH):
        gat[g % _NBUF].wait()
        scat[g % _NBUF] = start_scatter(g)
        nxt = g + _NBUF
        if nxt < _NCH:
            scat[nxt % _NBUF].wait()
            gat[nxt % _NBUF] = start_gather(nxt)
            scat[nxt % _NBUF] = None
    for s in scat:
        if s is not None:
            s.wait()


_SC_ROWS = 3072


def _sc_top_body(enc_hbm, out_hbm, *scratch):
    bufs = scratch[:_NBUF]
    gsems = scratch[_NBUF:2 * _NBUF]
    ssems = scratch[2 * _NBUF:3 * _NBUF]
    wid = lax.axis_index("s") * _NC + lax.axis_index("c")
    rows_per_w = _SC_ROWS // _NW
    base = wid * rows_per_w
    chunks = []
    off = 0
    while off < rows_per_w:
        sz = min(_C, rows_per_w - off)
        chunks.append((off, sz))
        off += sz

    def start_gather(g):
        off, sz = chunks[g]
        return pltpu.async_copy(
            enc_hbm.at[pl.ds(base + off, sz)],
            bufs[g % _NBUF].at[pl.ds(0, sz)],
            gsems[g % _NBUF],
        )

    def start_scatter(g):
        off, sz = chunks[g]
        return pltpu.async_copy(
            bufs[g % _NBUF].at[pl.ds(0, sz)],
            out_hbm.at[pl.ds(base + off, sz)],
            ssems[g % _NBUF],
        )

    nch = len(chunks)
    scat = [None] * _NBUF
    gat = [None] * _NBUF
    for g in range(min(_NBUF, nch)):
        gat[g % _NBUF] = start_gather(g)
    for g in range(nch):
        gat[g % _NBUF].wait()
        scat[g % _NBUF] = start_scatter(g)
        nxt = g + _NBUF
        if nxt < nch:
            scat[nxt % _NBUF].wait()
            gat[nxt % _NBUF] = start_gather(nxt)
            scat[nxt % _NBUF] = None
    for s in scat:
        if s is not None:
            s.wait()


def _tc_copy_body(enc_ref, out_ref):
    out_ref[...] = enc_ref[...]


def kernel(x, encodings):
    seq, d = encodings.shape
    mesh = plsc.VectorSubcoreMesh(core_axis_name="c", subcore_axis_name="s")
    sc_part = pl.kernel(
        _sc_top_body,
        mesh=mesh,
        out_type=jax.ShapeDtypeStruct((_SC_ROWS, d), jnp.float32),
        scratch_types=(
            [pltpu.VMEM((_C, _D), jnp.float32)] * _NBUF
            + [pltpu.SemaphoreType.DMA] * (2 * _NBUF)
        ),
    )(encodings)
    tc_rows = seq - _SC_ROWS
    blk = 1024
    tc_part = pl.pallas_call(
        _tc_copy_body,
        grid=(tc_rows // blk,),
        in_specs=[pl.BlockSpec((blk, d), lambda i: (i + _SC_ROWS // blk, 0))],
        out_specs=pl.BlockSpec((blk, d), lambda i: (i, 0)),
        out_shape=jax.ShapeDtypeStruct((tc_rows, d), jnp.float32),
    )(encodings)
    return jnp.concatenate([sc_part, tc_part], axis=0)[None, :, :]


# speedup vs baseline: 1.5339x; 1.5339x over previous
import jax
import jax.numpy as jnp
from jax import lax
from jax.experimental import pallas as pl
from jax.experimental.pallas import tpu as pltpu
from jax.experimental.pallas import tpu_sc as plsc

_SEQ, _D = 8192, 2048
_NSC = 2
_ROWS_PER_SC = _SEQ // _NSC
_C = 128
_NCH = _ROWS_PER_SC // _C
_NBUF = 4
_LEAD = 2


def _scs_body(enc_hbm, out_hbm, *scratch):
    bufs = scratch[:_NBUF]
    gsems = scratch[_NBUF:2 * _NBUF]
    ssems = scratch[2 * _NBUF:3 * _NBUF]
    cid = lax.axis_index("c")
    base = cid * _ROWS_PER_SC

    def start_gather(g):
        return pltpu.async_copy(
            enc_hbm.at[pl.ds(base + g * _C, _C)], bufs[g % _NBUF], gsems[g % _NBUF]
        )

    def start_scatter(g):
        return pltpu.async_copy(
            bufs[g % _NBUF], out_hbm.at[pl.ds(base + g * _C, _C)], ssems[g % _NBUF]
        )

    gat = [None] * _NCH
    scat = [None] * _NCH
    for j in range(_LEAD):
        gat[j] = start_gather(j)
    for g in range(_NCH):
        j = g + _LEAD
        if j < _NCH:
            jn = j - _NBUF
            if jn >= 0:
                scat[jn].wait()
            gat[j] = start_gather(j)
        gat[g].wait()
        scat[g] = start_scatter(g)
    for g in range(max(0, _NCH - _NBUF), _NCH):
        scat[g].wait()


def kernel(x, encodings):
    seq, d = encodings.shape
    mesh = plsc.ScalarSubcoreMesh(axis_name="c", num_cores=_NSC)
    out = pl.kernel(
        _scs_body,
        mesh=mesh,
        out_type=jax.ShapeDtypeStruct((seq, d), jnp.float32),
        scratch_types=(
            [pltpu.VMEM_SHARED((_C, _D), jnp.float32)] * _NBUF
            + [pltpu.SemaphoreType.DMA] * (2 * _NBUF)
        ),
    )(encodings)
    return out[None, :, :]
